# baseline (speedup 1.0000x reference)
import jax
import jax.numpy as jnp
from jax.experimental import pallas as pl
from jax.experimental.pallas import tpu as pltpu

T = 8192
D_IN = 4096
D_OUT = 4096
R = 16
E = 8
ER = E * R
_SCALING = 32.0 / 16.0

BTA = 1024
BTB = 2048
BD = 512


def _gate_body(x_ref, rw_ref, aall_ref, logits_ref, aw_ref, xb_ref):
    x = x_ref[...]
    logits = jax.lax.dot_general(
        x, rw_ref[...], (((1,), (1,)), ((), ())),
        preferred_element_type=jnp.float32)
    logits_ref[...] = logits
    iota_e = jax.lax.broadcasted_iota(jnp.int32, (BTA, E), 1)
    v1 = jnp.max(logits, axis=1, keepdims=True)
    i1 = jnp.min(jnp.where(logits == v1, iota_e, E), axis=1, keepdims=True)
    masked = jnp.where(iota_e == i1, -jnp.inf, logits)
    v2 = jnp.max(masked, axis=1, keepdims=True)
    i2 = jnp.min(jnp.where(masked == v2, iota_e, E), axis=1, keepdims=True)
    ed = jnp.exp(v2 - v1)
    denom = 1.0 + ed
    g1 = 1.0 / denom
    g2 = ed / denom
    lane_e = jax.lax.broadcasted_iota(jnp.int32, (BTA, ER), 1) // R
    gate_x = (jnp.where(lane_e == i1, g1, 0.0)
              + jnp.where(lane_e == i2, g2, 0.0)) * _SCALING
    a = jax.lax.dot_general(
        x, aall_ref[...], (((1,), (1,)), ((), ())),
        preferred_element_type=jnp.float32)
    aw_ref[...] = (a * gate_x).astype(jnp.bfloat16)
    xb_ref[...] = x.astype(jnp.bfloat16)


def _main_body(x_ref, w_ref, b_ref, aw_ref, ball_ref, out_ref):
    acc = jax.lax.dot_general(
        x_ref[...], w_ref[...], (((1,), (1,)), ((), ())),
        preferred_element_type=jnp.float32)
    acc += jnp.dot(aw_ref[...], ball_ref[...],
                   preferred_element_type=jnp.float32)
    out_ref[...] = acc + b_ref[...]


def kernel(x, base_W, base_b, router_W, lora_A, lora_B):
    a_all = lora_A.reshape(ER, D_IN)
    b_all = jnp.transpose(lora_B, (0, 2, 1)).reshape(ER, D_OUT)
    b_all = b_all.astype(jnp.bfloat16)
    w_bf = base_W.astype(jnp.bfloat16)
    bias = base_b.reshape(1, D_OUT)

    logits, aw, xb = pl.pallas_call(
        _gate_body,
        grid=(T // BTA,),
        in_specs=[
            pl.BlockSpec((BTA, D_IN), lambda i: (i, 0)),
            pl.BlockSpec((E, D_IN), lambda i: (0, 0)),
            pl.BlockSpec((ER, D_IN), lambda i: (0, 0)),
        ],
        out_specs=[
            pl.BlockSpec((BTA, E), lambda i: (i, 0)),
            pl.BlockSpec((BTA, ER), lambda i: (i, 0)),
            pl.BlockSpec((BTA, D_IN), lambda i: (i, 0)),
        ],
        out_shape=[
            jax.ShapeDtypeStruct((T, E), jnp.float32),
            jax.ShapeDtypeStruct((T, ER), jnp.bfloat16),
            jax.ShapeDtypeStruct((T, D_IN), jnp.bfloat16),
        ],
        compiler_params=pltpu.CompilerParams(
            dimension_semantics=("parallel",)),
    )(x, router_W, a_all)

    out = pl.pallas_call(
        _main_body,
        grid=(T // BTB, D_OUT // BD),
        in_specs=[
            pl.BlockSpec((BTB, D_IN), lambda i, j: (i, 0)),
            pl.BlockSpec((BD, D_IN), lambda i, j: (j, 0)),
            pl.BlockSpec((1, BD), lambda i, j: (0, j)),
            pl.BlockSpec((BTB, ER), lambda i, j: (i, 0)),
            pl.BlockSpec((ER, BD), lambda i, j: (0, j)),
        ],
        out_specs=pl.BlockSpec((BTB, BD), lambda i, j: (i, j)),
        out_shape=jax.ShapeDtypeStruct((T, D_OUT), jnp.float32),
        compiler_params=pltpu.CompilerParams(
            dimension_semantics=("parallel", "arbitrary")),
    )(xb, w_bf, bias, aw, b_all)
    return out, logits
